# baseline (speedup 1.0000x reference)
import functools

import jax
import jax.numpy as jnp
from jax import lax
from jax.experimental import pallas as pl
from jax.experimental.pallas import tpu as pltpu

_W1 = 1.0
_W2 = 1.0
_S1 = 0.5
_S2 = 0.5

_B = 16384
_C = 1000
_BR = 512
_NBLK = _B // _BR


def _tc_body(pred_ref, tgt_ref, marg_ref, out_ref):
    p = pred_ref[...]
    t = tgt_ref[...]
    m = marg_ref[...]
    cols = jax.lax.broadcasted_iota(jnp.int32, (_BR, _C), 1)
    n = jnp.sum(jnp.where(cols == t, p, 0.0), axis=1, keepdims=True)
    w1 = _W1 * jnp.exp(-_S1 * m * m)
    w2 = _W2 * jnp.exp(-_S2 * m * m)
    contrib = jnp.where(m > 0, w1 * n, 0.0) + jnp.where(m < 0, w2 * n, 0.0)
    out_ref[...] = jnp.broadcast_to(jnp.sum(contrib), (1, 1, 128))


_tc_partials = pl.pallas_call(
    _tc_body,
    out_shape=jax.ShapeDtypeStruct((_NBLK, 1, 128), jnp.float32),
    grid=(_NBLK,),
    in_specs=[
        pl.BlockSpec((_BR, _C), lambda i: (i, 0)),
        pl.BlockSpec((_BR, 1), lambda i: (i, 0)),
        pl.BlockSpec((_BR, 1), lambda i: (i, 0)),
    ],
    out_specs=pl.BlockSpec((1, 1, 128), lambda i: (i, 0, 0)),
)


def kernel(preds, targets, margin):
    partials = _tc_partials(preds, targets[:, None], margin[:, None])
    return -jnp.sum(partials[:, 0, 0]) / margin.shape[0]
